# baseline (speedup 1.0000x reference)
import functools

import jax
import jax.numpy as jnp
from jax import lax
from jax.experimental import pallas as pl
from jax.experimental.pallas import tpu as pltpu
from jax.experimental.pallas import tpu_sc as plsc

_NUM_CORES = 2
_NUM_SUBCORES = 16
_NW = _NUM_CORES * _NUM_SUBCORES


def _sc_gather(means, question, idx):
    B = idx.shape[0]
    D = means.shape[1]
    b_per_w = B // _NW
    assert B % (8 * _NW) == 0

    mesh = plsc.VectorSubcoreMesh(core_axis_name="c", subcore_axis_name="s")

    @functools.partial(
        pl.kernel,
        out_type=(
            jax.ShapeDtypeStruct((B, D), jnp.float32),
            jax.ShapeDtypeStruct((B, D), jnp.float32),
        ),
        mesh=mesh,
        scratch_types=[
            pltpu.VMEM((b_per_w,), jnp.int32),
            pltpu.VMEM((b_per_w, D), jnp.float32),
            pltpu.VMEM((b_per_w, D), jnp.float32),
            pltpu.SemaphoreType.DMA,
            pltpu.SemaphoreType.DMA,
        ],
        compiler_params=pltpu.CompilerParams(use_tc_tiling_on_sc=False),
    )
    def gather_kernel(means_hbm, question_hbm, idx_hbm, gm_hbm, gq_hbm,
                      idx_v, m_v, q_v, sem_m, sem_q):
        wid = lax.axis_index("s") * _NUM_CORES + lax.axis_index("c")
        base = wid * b_per_w
        pltpu.sync_copy(idx_hbm.at[pl.ds(base, b_per_w)], idx_v)
        cm = pltpu.async_copy(means_hbm.at[idx_v], m_v, sem_m)
        cq = pltpu.async_copy(question_hbm.at[idx_v], q_v, sem_q)
        cm.wait()
        cq.wait()
        pltpu.sync_copy(m_v, gm_hbm.at[pl.ds(base, b_per_w)])
        pltpu.sync_copy(q_v, gq_hbm.at[pl.ds(base, b_per_w)])

    return gather_kernel(means, question, idx)


def _tc_subtract(prosody, gm, gq, block_b):
    B, T, D = prosody.shape

    def body(p_ref, m_ref, q_ref, o_ref):
        c = (m_ref[...] + q_ref[...]) * 0.5
        o_ref[...] = p_ref[...] - c[:, None, :]

    return pl.pallas_call(
        body,
        grid=(B // block_b,),
        in_specs=[
            pl.BlockSpec((block_b, T, D), lambda i: (i, 0, 0)),
            pl.BlockSpec((block_b, D), lambda i: (i, 0)),
            pl.BlockSpec((block_b, D), lambda i: (i, 0)),
        ],
        out_specs=pl.BlockSpec((block_b, T, D), lambda i: (i, 0, 0)),
        out_shape=jax.ShapeDtypeStruct((B, T, D), jnp.float32),
    )(prosody, gm, gq)


def kernel(prosody, spkr_id, means, question):
    idx = spkr_id.astype(jnp.int32)
    gm, gq = _sc_gather(means, question, idx)
    return _tc_subtract(prosody, gm, gq, block_b=64)


# speedup vs baseline: 1.0490x; 1.0490x over previous
import functools

import jax
import jax.numpy as jnp
from jax import lax
from jax.experimental import pallas as pl
from jax.experimental.pallas import tpu as pltpu
from jax.experimental.pallas import tpu_sc as plsc

_NUM_CORES = 2
_NUM_SUBCORES = 16
_NW = _NUM_CORES * _NUM_SUBCORES


def _sc_gather(means, question, idx):
    B = idx.shape[0]
    D = means.shape[1]
    b_per_w = B // _NW
    assert B % (8 * _NW) == 0

    mesh = plsc.VectorSubcoreMesh(core_axis_name="c", subcore_axis_name="s")

    @functools.partial(
        pl.kernel,
        out_type=(
            jax.ShapeDtypeStruct((B, D), jnp.float32),
            jax.ShapeDtypeStruct((B, D), jnp.float32),
        ),
        mesh=mesh,
        scratch_types=[
            pltpu.VMEM((b_per_w,), jnp.int32),
            pltpu.VMEM((b_per_w, D), jnp.float32),
            pltpu.VMEM((b_per_w, D), jnp.float32),
            pltpu.SemaphoreType.DMA,
            pltpu.SemaphoreType.DMA,
        ],
        compiler_params=pltpu.CompilerParams(use_tc_tiling_on_sc=False),
    )
    def gather_kernel(means_hbm, question_hbm, idx_hbm, gm_hbm, gq_hbm,
                      idx_v, m_v, q_v, sem_m, sem_q):
        wid = lax.axis_index("s") * _NUM_CORES + lax.axis_index("c")
        base = wid * b_per_w
        pltpu.sync_copy(idx_hbm.at[pl.ds(base, b_per_w)], idx_v)
        cm = pltpu.async_copy(means_hbm.at[idx_v], m_v, sem_m)
        cq = pltpu.async_copy(question_hbm.at[idx_v], q_v, sem_q)
        cm.wait()
        cq.wait()
        pltpu.sync_copy(m_v, gm_hbm.at[pl.ds(base, b_per_w)])
        pltpu.sync_copy(q_v, gq_hbm.at[pl.ds(base, b_per_w)])

    return gather_kernel(means, question, idx)


def _tc_subtract(prosody, gm, gq, n_chunks, depth):
    B, T, D = prosody.shape
    ch = B // n_chunks

    def body(p_hbm, gm_hbm, gq_hbm, o_hbm, pbuf, obuf, gmv, gqv,
             in_sems, out_sems, gm_sem, gq_sem):
        gm_cp = pltpu.make_async_copy(gm_hbm, gmv, gm_sem)
        gq_cp = pltpu.make_async_copy(gq_hbm, gqv, gq_sem)
        gm_cp.start()
        gq_cp.start()

        in_cps = [
            pltpu.make_async_copy(
                p_hbm.at[pl.ds(j * ch, ch)], pbuf.at[j % depth],
                in_sems.at[j % depth])
            for j in range(n_chunks)
        ]
        out_cps = [
            pltpu.make_async_copy(
                obuf.at[j % depth], o_hbm.at[pl.ds(j * ch, ch)],
                out_sems.at[j % depth])
            for j in range(n_chunks)
        ]
        for j in range(depth):
            in_cps[j].start()
        gm_cp.wait()
        gq_cp.wait()
        for j in range(n_chunks):
            in_cps[j].wait()
            if j >= depth:
                out_cps[j - depth].wait()
            cs = (gmv[pl.ds(j * ch, ch), :] + gqv[pl.ds(j * ch, ch), :]) * 0.5
            obuf[j % depth] = pbuf[j % depth] - cs[:, None, :]
            out_cps[j].start()
            if j + depth < n_chunks:
                in_cps[j + depth].start()
        for j in range(n_chunks - depth, n_chunks):
            out_cps[j].wait()

    return pl.pallas_call(
        body,
        in_specs=[
            pl.BlockSpec(memory_space=pl.ANY),
            pl.BlockSpec(memory_space=pl.ANY),
            pl.BlockSpec(memory_space=pl.ANY),
        ],
        out_specs=pl.BlockSpec(memory_space=pl.ANY),
        out_shape=jax.ShapeDtypeStruct((B, T, D), jnp.float32),
        scratch_shapes=[
            pltpu.VMEM((depth, ch, T, D), jnp.float32),
            pltpu.VMEM((depth, ch, T, D), jnp.float32),
            pltpu.VMEM((B, D), jnp.float32),
            pltpu.VMEM((B, D), jnp.float32),
            pltpu.SemaphoreType.DMA((depth,)),
            pltpu.SemaphoreType.DMA((depth,)),
            pltpu.SemaphoreType.DMA,
            pltpu.SemaphoreType.DMA,
        ],
    )(prosody, gm, gq)


def kernel(prosody, spkr_id, means, question):
    idx = spkr_id.astype(jnp.int32)
    gm, gq = _sc_gather(means, question, idx)
    return _tc_subtract(prosody, gm, gq, n_chunks=32, depth=4)


# speedup vs baseline: 2.1985x; 2.0958x over previous
import functools

import jax
import jax.numpy as jnp
from jax import lax
from jax.experimental import pallas as pl
from jax.experimental.pallas import tpu as pltpu
from jax.experimental.pallas import tpu_sc as plsc

_NUM_CORES = 2
_NUM_SUBCORES = 16
_NW = _NUM_CORES * _NUM_SUBCORES


def _sc_gather(means, question, idx):
    B = idx.shape[0]
    D = means.shape[1]
    b_per_w = B // _NW
    assert B % (8 * _NW) == 0

    mesh = plsc.VectorSubcoreMesh(core_axis_name="c", subcore_axis_name="s")

    @functools.partial(
        pl.kernel,
        out_type=(
            jax.ShapeDtypeStruct((B, D), jnp.float32),
            jax.ShapeDtypeStruct((B, D), jnp.float32),
        ),
        mesh=mesh,
        scratch_types=[
            pltpu.VMEM((b_per_w,), jnp.int32),
            pltpu.VMEM((b_per_w, D), jnp.float32),
            pltpu.VMEM((b_per_w, D), jnp.float32),
            pltpu.SemaphoreType.DMA,
            pltpu.SemaphoreType.DMA,
        ],
        compiler_params=pltpu.CompilerParams(use_tc_tiling_on_sc=False),
    )
    def gather_kernel(means_hbm, question_hbm, idx_hbm, gm_hbm, gq_hbm,
                      idx_v, m_v, q_v, sem_m, sem_q):
        wid = lax.axis_index("s") * _NUM_CORES + lax.axis_index("c")
        base = wid * b_per_w
        pltpu.sync_copy(idx_hbm.at[pl.ds(base, b_per_w)], idx_v)
        cm = pltpu.async_copy(means_hbm.at[idx_v], m_v, sem_m)
        cq = pltpu.async_copy(question_hbm.at[idx_v], q_v, sem_q)
        cm.wait()
        cq.wait()
        pltpu.sync_copy(m_v, gm_hbm.at[pl.ds(base, b_per_w)])
        pltpu.sync_copy(q_v, gq_hbm.at[pl.ds(base, b_per_w)])

    return gather_kernel(means, question, idx)


def _tc_subtract_t(pt, gmt, gqt, n_chunks, depth):
    T, D, B = pt.shape
    ch = T // n_chunks

    def body(p_hbm, gmt_hbm, gqt_hbm, o_hbm, pbuf, obuf, cbuf,
             in_sems, out_sems, gm_sem, gq_sem):
        gm_cp = pltpu.make_async_copy(gmt_hbm, cbuf.at[0], gm_sem)
        gq_cp = pltpu.make_async_copy(gqt_hbm, cbuf.at[1], gq_sem)
        gm_cp.start()
        gq_cp.start()

        in_cps = [
            pltpu.make_async_copy(
                p_hbm.at[pl.ds(j * ch, ch)], pbuf.at[j % depth],
                in_sems.at[j % depth])
            for j in range(n_chunks)
        ]
        out_cps = [
            pltpu.make_async_copy(
                obuf.at[j % depth], o_hbm.at[pl.ds(j * ch, ch)],
                out_sems.at[j % depth])
            for j in range(n_chunks)
        ]
        for j in range(depth):
            in_cps[j].start()
        gm_cp.wait()
        gq_cp.wait()
        c = (cbuf[0] + cbuf[1]) * 0.5
        for j in range(n_chunks):
            in_cps[j].wait()
            if j >= depth:
                out_cps[j - depth].wait()
            obuf[j % depth] = pbuf[j % depth] - c[None, :, :]
            out_cps[j].start()
            if j + depth < n_chunks:
                in_cps[j + depth].start()
        for j in range(n_chunks - depth, n_chunks):
            out_cps[j].wait()

    return pl.pallas_call(
        body,
        in_specs=[
            pl.BlockSpec(memory_space=pl.ANY),
            pl.BlockSpec(memory_space=pl.ANY),
            pl.BlockSpec(memory_space=pl.ANY),
        ],
        out_specs=pl.BlockSpec(memory_space=pl.ANY),
        out_shape=jax.ShapeDtypeStruct((T, D, B), jnp.float32),
        scratch_shapes=[
            pltpu.VMEM((depth, ch, D, B), jnp.float32),
            pltpu.VMEM((depth, ch, D, B), jnp.float32),
            pltpu.VMEM((2, D, B), jnp.float32),
            pltpu.SemaphoreType.DMA((depth,)),
            pltpu.SemaphoreType.DMA((depth,)),
            pltpu.SemaphoreType.DMA,
            pltpu.SemaphoreType.DMA,
        ],
    )(pt, gmt, gqt)


def kernel(prosody, spkr_id, means, question):
    idx = spkr_id.astype(jnp.int32)
    gm, gq = _sc_gather(means, question, idx)
    pt = jnp.transpose(prosody, (1, 2, 0))
    gmt = jnp.transpose(gm, (1, 0))
    gqt = jnp.transpose(gq, (1, 0))
    out_t = _tc_subtract_t(pt, gmt, gqt, n_chunks=10, depth=4)
    return jnp.transpose(out_t, (2, 0, 1))


# speedup vs baseline: 4.3721x; 1.9887x over previous
import functools

import jax
import jax.numpy as jnp
from jax import lax
from jax.experimental import pallas as pl
from jax.experimental.pallas import tpu as pltpu
from jax.experimental.pallas import tpu_sc as plsc

_NUM_CORES = 2
_NUM_SUBCORES = 16
_NW = _NUM_CORES * _NUM_SUBCORES


def _tc_pack_sum(mt, qt, half, block_k):
    D, V = mt.shape
    grid = half // block_k
    off = half // block_k

    def body(ma_ref, qa_ref, mb_ref, qb_ref, o_ref):
        ta = jnp.transpose(ma_ref[...] + qa_ref[...], (1, 0))
        tb = jnp.transpose(mb_ref[...] + qb_ref[...], (1, 0))
        o_ref[...] = jnp.concatenate([ta, tb], axis=1)

    return pl.pallas_call(
        body,
        grid=(grid,),
        in_specs=[
            pl.BlockSpec((D, block_k), lambda i: (0, i)),
            pl.BlockSpec((D, block_k), lambda i: (0, i)),
            pl.BlockSpec((D, block_k), lambda i: (0, i + off)),
            pl.BlockSpec((D, block_k), lambda i: (0, i + off)),
        ],
        out_specs=pl.BlockSpec((block_k, 2 * D), lambda i: (i, 0)),
        out_shape=jax.ShapeDtypeStruct((half, 2 * D), jnp.float32),
    )(mt, qt, mt, qt)


def _sc_gather_packed(sum2, idx, half):
    B = idx.shape[0]
    L2 = sum2.shape[1]
    b_per_w = B // _NW
    assert B % (8 * _NW) == 0

    mesh = plsc.VectorSubcoreMesh(core_axis_name="c", subcore_axis_name="s")

    @functools.partial(
        pl.kernel,
        out_type=jax.ShapeDtypeStruct((B, L2), jnp.float32),
        mesh=mesh,
        scratch_types=[
            pltpu.VMEM((b_per_w,), jnp.int32),
            pltpu.VMEM((b_per_w,), jnp.int32),
            pltpu.VMEM((b_per_w, L2), jnp.float32),
            pltpu.SemaphoreType.DMA,
        ],
    )
    def gather_kernel(sum2_hbm, idx_hbm, g2_hbm, idx_v, idx2_v, g_v, sem):
        wid = lax.axis_index("s") * _NUM_CORES + lax.axis_index("c")
        base = wid * b_per_w
        pltpu.sync_copy(idx_hbm.at[pl.ds(base, b_per_w)], idx_v)
        for i in range(b_per_w // 16):
            v = idx_v[pl.ds(i * 16, 16)]
            idx2_v[pl.ds(i * 16, 16)] = jnp.where(v >= half, v - half, v)
        pltpu.async_copy(sum2_hbm.at[idx2_v], g_v, sem).wait()
        pltpu.sync_copy(g_v, g2_hbm.at[pl.ds(base, b_per_w)])

    return gather_kernel(sum2, idx)


def _tc_subtract_t(pt, g2, idx, half, n_chunks, depth):
    T, D, B = pt.shape
    ch = T // n_chunks

    def body(p_hbm, g2_ref, idx_ref, o_hbm, pbuf, obuf, in_sems, out_sems):
        par = idx_ref[...][:, None]
        sel = jnp.where(par >= half, g2_ref[:, D:2 * D], g2_ref[:, 0:D])
        c = jnp.transpose(sel * 0.5, (1, 0))

        in_cps = [
            pltpu.make_async_copy(
                p_hbm.at[pl.ds(j * ch, ch)], pbuf.at[j % depth],
                in_sems.at[j % depth])
            for j in range(n_chunks)
        ]
        out_cps = [
            pltpu.make_async_copy(
                obuf.at[j % depth], o_hbm.at[pl.ds(j * ch, ch)],
                out_sems.at[j % depth])
            for j in range(n_chunks)
        ]
        for j in range(depth):
            in_cps[j].start()
        for j in range(n_chunks):
            in_cps[j].wait()
            if j >= depth:
                out_cps[j - depth].wait()
            obuf[j % depth] = pbuf[j % depth] - c[None, :, :]
            out_cps[j].start()
            if j + depth < n_chunks:
                in_cps[j + depth].start()
        for j in range(n_chunks - depth, n_chunks):
            out_cps[j].wait()

    return pl.pallas_call(
        body,
        in_specs=[
            pl.BlockSpec(memory_space=pl.ANY),
            pl.BlockSpec((B, 2 * D), lambda: (0, 0)),
            pl.BlockSpec((B,), lambda: (0,)),
        ],
        out_specs=pl.BlockSpec(memory_space=pl.ANY),
        out_shape=jax.ShapeDtypeStruct((T, D, B), jnp.float32),
        scratch_shapes=[
            pltpu.VMEM((depth, ch, D, B), jnp.float32),
            pltpu.VMEM((depth, ch, D, B), jnp.float32),
            pltpu.SemaphoreType.DMA((depth,)),
            pltpu.SemaphoreType.DMA((depth,)),
        ],
    )(pt, g2, idx)


def kernel(prosody, spkr_id, means, question):
    idx = spkr_id.astype(jnp.int32)
    mt = jnp.transpose(means, (1, 0))
    qt = jnp.transpose(question, (1, 0))
    half = 51200
    sum2 = _tc_pack_sum(mt, qt, half=half, block_k=3200)
    g2 = _sc_gather_packed(sum2, idx, half=half)
    pt = jnp.transpose(prosody, (1, 2, 0))
    out_t = _tc_subtract_t(pt, g2, idx, half=half, n_chunks=10, depth=4)
    return jnp.transpose(out_t, (2, 0, 1))
